# baseline (speedup 1.0000x reference)
import jax
import jax.numpy as jnp
from jax.experimental import pallas as pl
from jax.experimental.pallas import tpu as pltpu

_N_ART = 50000
_ROW_BLOCK = 25000


def _fused_body(x_ref, w_ref, b_ref, lw_ref, lb_ref, o_ref):
    x = x_ref[...]
    pre = jnp.dot(x, w_ref[...], preferred_element_type=jnp.float32)
    pre = pre + b_ref[...]
    gi = jax.nn.sigmoid(pre[:, 0:64])
    gc = jnp.tanh(pre[:, 64:128])
    go = jax.nn.sigmoid(pre[:, 128:192])
    h = go * jnp.tanh(gi * gc)
    h = jnp.maximum(h, 0.0)
    o_ref[...] = (
        jnp.dot(h, lw_ref[...], preferred_element_type=jnp.float32)
        + lb_ref[...]
    )


def kernel(x_article, x_author, params, edge_index_cites, edge_index_writes,
           edge_index_written_by):
    del x_author, edge_index_cites, edge_index_writes, edge_index_written_by
    p = params
    w_cat = jnp.concatenate(
        [p["W_i_article"], p["W_c_article"], p["W_o_article"]], axis=1)
    b_parts = []
    for g in ("i", "c", "o"):
        b_parts.append(p["b_%s_article" % g][0]
                       + p["conv_%s_cites_bl" % g]
                       + p["conv_%s_writes_bl" % g])
    b_cat = jnp.concatenate(b_parts)[None, :]
    lin_b = p["lin_b"][None, :]

    n_blocks = _N_ART // _ROW_BLOCK
    out = pl.pallas_call(
        _fused_body,
        grid=(n_blocks,),
        in_specs=[
            pl.BlockSpec((_ROW_BLOCK, 128), lambda i: (i, 0)),
            pl.BlockSpec((128, 192), lambda i: (0, 0)),
            pl.BlockSpec((1, 192), lambda i: (0, 0)),
            pl.BlockSpec((64, 16), lambda i: (0, 0)),
            pl.BlockSpec((1, 16), lambda i: (0, 0)),
        ],
        out_specs=pl.BlockSpec((_ROW_BLOCK, 16), lambda i: (i, 0)),
        out_shape=jax.ShapeDtypeStruct((_N_ART, 16), jnp.float32),
        compiler_params=pltpu.CompilerParams(
            dimension_semantics=("parallel",)),
    )(x_article, w_cat, b_cat, p["lin_W"], lin_b)
    return out


# speedup vs baseline: 1.0270x; 1.0270x over previous
import jax
import jax.numpy as jnp
from jax.experimental import pallas as pl
from jax.experimental.pallas import tpu as pltpu

_N_ART = 50000
_ROW_BLOCK = 5000


def _fused_body(x_ref, w_ref, b_ref, lw_ref, lb_ref, o_ref):
    x = x_ref[...]
    pre = jnp.dot(x, w_ref[...], preferred_element_type=jnp.float32)
    pre = pre + b_ref[...]
    gi = jax.nn.sigmoid(pre[:, 0:64])
    gc = jnp.tanh(pre[:, 64:128])
    go = jax.nn.sigmoid(pre[:, 128:192])
    h = go * jnp.tanh(gi * gc)
    h = jnp.maximum(h, 0.0)
    o_ref[...] = (
        jnp.dot(h, lw_ref[...], preferred_element_type=jnp.float32)
        + lb_ref[...]
    )


def kernel(x_article, x_author, params, edge_index_cites, edge_index_writes,
           edge_index_written_by):
    del x_author, edge_index_cites, edge_index_writes, edge_index_written_by
    p = params
    w_cat = jnp.concatenate(
        [p["W_i_article"], p["W_c_article"], p["W_o_article"]], axis=1)
    b_parts = []
    for g in ("i", "c", "o"):
        b_parts.append(p["b_%s_article" % g][0]
                       + p["conv_%s_cites_bl" % g]
                       + p["conv_%s_writes_bl" % g])
    b_cat = jnp.concatenate(b_parts)[None, :]
    lin_b = p["lin_b"][None, :]

    n_blocks = _N_ART // _ROW_BLOCK
    out = pl.pallas_call(
        _fused_body,
        grid=(n_blocks,),
        in_specs=[
            pl.BlockSpec((_ROW_BLOCK, 128), lambda i: (i, 0)),
            pl.BlockSpec((128, 192), lambda i: (0, 0)),
            pl.BlockSpec((1, 192), lambda i: (0, 0)),
            pl.BlockSpec((64, 16), lambda i: (0, 0)),
            pl.BlockSpec((1, 16), lambda i: (0, 0)),
        ],
        out_specs=pl.BlockSpec((_ROW_BLOCK, 16), lambda i: (i, 0)),
        out_shape=jax.ShapeDtypeStruct((_N_ART, 16), jnp.float32),
        compiler_params=pltpu.CompilerParams(
            dimension_semantics=("parallel",)),
    )(x_article, w_cat, b_cat, p["lin_W"], lin_b)
    return out


# speedup vs baseline: 1.0422x; 1.0148x over previous
import jax
import jax.numpy as jnp
from jax.experimental import pallas as pl
from jax.experimental.pallas import tpu as pltpu

_N_ART = 50000
_ROW_BLOCK = 10000


def _fused_body(x_ref, wi_ref, wc_ref, wo_ref, b_ref, lw_ref, lb_ref, o_ref):
    x = x_ref[...]
    b = b_ref[...]
    pre_i = jnp.dot(x, wi_ref[...], preferred_element_type=jnp.float32)
    pre_c = jnp.dot(x, wc_ref[...], preferred_element_type=jnp.float32)
    pre_o = jnp.dot(x, wo_ref[...], preferred_element_type=jnp.float32)
    gi = jax.nn.sigmoid(pre_i + b[0:1, :])
    gc = jnp.tanh(pre_c + b[1:2, :])
    go = jax.nn.sigmoid(pre_o + b[2:3, :])
    h = go * jnp.tanh(gi * gc)
    h = jnp.maximum(h, 0.0)
    o_ref[...] = (
        jnp.dot(h, lw_ref[...], preferred_element_type=jnp.float32)
        + lb_ref[...]
    )


def kernel(x_article, x_author, params, edge_index_cites, edge_index_writes,
           edge_index_written_by):
    del x_author, edge_index_cites, edge_index_writes, edge_index_written_by
    p = params
    b_cat = jnp.stack(
        [p["b_%s_article" % g][0]
         + p["conv_%s_cites_bl" % g]
         + p["conv_%s_writes_bl" % g]
         for g in ("i", "c", "o")])
    lin_b = p["lin_b"][None, :]

    n_blocks = _N_ART // _ROW_BLOCK
    out = pl.pallas_call(
        _fused_body,
        grid=(n_blocks,),
        in_specs=[
            pl.BlockSpec((_ROW_BLOCK, 128), lambda i: (i, 0)),
            pl.BlockSpec((128, 64), lambda i: (0, 0)),
            pl.BlockSpec((128, 64), lambda i: (0, 0)),
            pl.BlockSpec((128, 64), lambda i: (0, 0)),
            pl.BlockSpec((3, 64), lambda i: (0, 0)),
            pl.BlockSpec((64, 16), lambda i: (0, 0)),
            pl.BlockSpec((1, 16), lambda i: (0, 0)),
        ],
        out_specs=pl.BlockSpec((_ROW_BLOCK, 16), lambda i: (i, 0)),
        out_shape=jax.ShapeDtypeStruct((_N_ART, 16), jnp.float32),
        compiler_params=pltpu.CompilerParams(
            dimension_semantics=("parallel",)),
    )(x_article, p["W_i_article"], p["W_c_article"], p["W_o_article"],
      b_cat, p["lin_W"], lin_b)
    return out


# speedup vs baseline: 1.1387x; 1.0927x over previous
import jax
import jax.numpy as jnp
from jax.experimental import pallas as pl
from jax.experimental.pallas import tpu as pltpu

_N_ART = 50000
_ROW_BLOCK = 10000


def _sigmoid(z):
    return 0.5 + 0.5 * jnp.tanh(0.5 * z)


def _fused_body(x_ref, wi_ref, wc_ref, wo_ref,
                bi_ref, bc_ref, bo_ref,
                bli_c_ref, bli_w_ref, blc_c_ref, blc_w_ref,
                blo_c_ref, blo_w_ref,
                lw_ref, lb_ref, o_ref):
    x = x_ref[...]
    bi = bi_ref[...] + bli_c_ref[...] + bli_w_ref[...]
    bc = bc_ref[...] + blc_c_ref[...] + blc_w_ref[...]
    bo = bo_ref[...] + blo_c_ref[...] + blo_w_ref[...]
    pre_i = jnp.dot(x, wi_ref[...], preferred_element_type=jnp.float32)
    pre_c = jnp.dot(x, wc_ref[...], preferred_element_type=jnp.float32)
    pre_o = jnp.dot(x, wo_ref[...], preferred_element_type=jnp.float32)
    gi = _sigmoid(pre_i + bi)
    gc = jnp.tanh(pre_c + bc)
    go = _sigmoid(pre_o + bo)
    h = go * jnp.tanh(gi * gc)
    h = jnp.maximum(h, 0.0)
    o_ref[...] = (
        jnp.dot(h, lw_ref[...], preferred_element_type=jnp.float32)
        + lb_ref[...]
    )


def kernel(x_article, x_author, params, edge_index_cites, edge_index_writes,
           edge_index_written_by):
    del x_author, edge_index_cites, edge_index_writes, edge_index_written_by
    p = params

    def v(a):
        return a.reshape(1, -1)

    n_blocks = _N_ART // _ROW_BLOCK
    vec_spec = pl.BlockSpec((1, 64), lambda i: (0, 0))
    out = pl.pallas_call(
        _fused_body,
        grid=(n_blocks,),
        in_specs=[
            pl.BlockSpec((_ROW_BLOCK, 128), lambda i: (i, 0)),
            pl.BlockSpec((128, 64), lambda i: (0, 0)),
            pl.BlockSpec((128, 64), lambda i: (0, 0)),
            pl.BlockSpec((128, 64), lambda i: (0, 0)),
            vec_spec, vec_spec, vec_spec,
            vec_spec, vec_spec, vec_spec, vec_spec, vec_spec, vec_spec,
            pl.BlockSpec((64, 16), lambda i: (0, 0)),
            pl.BlockSpec((1, 16), lambda i: (0, 0)),
        ],
        out_specs=pl.BlockSpec((_ROW_BLOCK, 16), lambda i: (i, 0)),
        out_shape=jax.ShapeDtypeStruct((_N_ART, 16), jnp.float32),
        compiler_params=pltpu.CompilerParams(
            dimension_semantics=("parallel",)),
    )(x_article,
      p["W_i_article"], p["W_c_article"], p["W_o_article"],
      v(p["b_i_article"]), v(p["b_c_article"]), v(p["b_o_article"]),
      v(p["conv_i_cites_bl"]), v(p["conv_i_writes_bl"]),
      v(p["conv_c_cites_bl"]), v(p["conv_c_writes_bl"]),
      v(p["conv_o_cites_bl"]), v(p["conv_o_writes_bl"]),
      p["lin_W"], v(p["lin_b"]))
    return out
